# baseline (speedup 1.0000x reference)
import functools

import jax
import jax.numpy as jnp
from jax import lax
from jax.experimental import pallas as pl
from jax.experimental.pallas import tpu as pltpu
from jax.experimental.pallas import tpu_sc as plsc

_N = 10000
_E = 320000
_D = 128
_H = 128
_C = 40
_G = 64

_NP = 10112
_NC = 2
_NS = 16
_ROWS_PER_TILE = _NP // _NS
_EDGES_PER_TILE = _E // (_NC * _NS)
_KR = 100
_NCH = _EDGES_PER_TILE // _KR
_HH = 64

_MESH = plsc.VectorSubcoreMesh(
    core_axis_name="c", subcore_axis_name="s", num_cores=_NC, num_subcores=_NS
)


def _deg_body(dst_hbm, ones_hbm, zeros_hbm, out_hbm, acc, idx_d, ones_v, sem):
    cid = lax.axis_index("c")
    sid = lax.axis_index("s")
    wid = cid * _NS + sid
    r0 = sid * _ROWS_PER_TILE
    zcp = pltpu.async_copy(
        zeros_hbm.at[pl.ds(r0, _ROWS_PER_TILE)], acc.at[pl.ds(r0, _ROWS_PER_TILE)], sem
    )
    pltpu.sync_copy(dst_hbm.at[wid], idx_d)
    pltpu.sync_copy(ones_hbm, ones_v)
    zcp.wait()
    plsc.subcore_barrier()

    def chunk(j, carry):
        pltpu.sync_copy(ones_v, acc.at[idx_d.at[j]], add=True)
        return carry

    lax.fori_loop(0, _NCH, chunk, 0)
    plsc.subcore_barrier()
    pltpu.sync_copy(
        acc.at[pl.ds(r0, _ROWS_PER_TILE)],
        out_hbm.at[cid, pl.ds(r0, _ROWS_PER_TILE)],
    )


def _row_body(
    vals_hbm, src_hbm, dst_hbm, zeros_hbm, out_hbm,
    hws_sp, acc, idx_s, idx_d, buf, sem,
):
    cid = lax.axis_index("c")
    sid = lax.axis_index("s")
    wid = cid * _NS + sid
    r0 = sid * _ROWS_PER_TILE
    pltpu.sync_copy(src_hbm.at[wid], idx_s)
    pltpu.sync_copy(dst_hbm.at[wid], idx_d)
    for h in range(2):
        c0 = h * _HH
        zcp = pltpu.async_copy(
            zeros_hbm.at[pl.ds(r0, _ROWS_PER_TILE)],
            acc.at[pl.ds(r0, _ROWS_PER_TILE)],
            sem,
        )
        pltpu.sync_copy(
            vals_hbm.at[pl.ds(r0, _ROWS_PER_TILE), pl.ds(c0, _HH)],
            hws_sp.at[pl.ds(r0, _ROWS_PER_TILE)],
        )
        zcp.wait()
        plsc.subcore_barrier()

        def chunk(j, carry):
            pltpu.sync_copy(hws_sp.at[idx_s.at[j]], buf)
            pltpu.sync_copy(buf, acc.at[idx_d.at[j]], add=True)
            return carry

        lax.fori_loop(0, _NCH, chunk, 0)
        plsc.subcore_barrier()
        pltpu.sync_copy(
            acc.at[pl.ds(r0, _ROWS_PER_TILE)],
            out_hbm.at[cid, pl.ds(r0, _ROWS_PER_TILE), pl.ds(c0, _HH)],
        )


_SC_PARAMS = pltpu.CompilerParams(use_tc_tiling_on_sc=False)


def _deg_call(dst3, ones16, zeros16):
    return pl.kernel(
        _deg_body,
        out_type=jax.ShapeDtypeStruct((_NC, _NP, 16), jnp.float32),
        mesh=_MESH,
        compiler_params=_SC_PARAMS,
        scratch_types=[
            pltpu.VMEM_SHARED((_NP, 16), jnp.float32),
            pltpu.VMEM((_NCH, _KR), jnp.int32),
            pltpu.VMEM((_KR, 16), jnp.float32),
            pltpu.SemaphoreType.DMA,
        ],
    )(dst3, ones16, zeros16)


def _row_call(vals, src3, dst3, zeros):
    return pl.kernel(
        _row_body,
        out_type=jax.ShapeDtypeStruct((_NC, _NP, _H), jnp.float32),
        mesh=_MESH,
        compiler_params=_SC_PARAMS,
        scratch_types=[
            pltpu.VMEM_SHARED((_NP, _HH), jnp.float32),
            pltpu.VMEM_SHARED((_NP, _HH), jnp.float32),
            pltpu.VMEM((_NCH, _KR), jnp.int32),
            pltpu.VMEM((_NCH, _KR), jnp.int32),
            pltpu.VMEM((_KR, _HH), jnp.float32),
            pltpu.SemaphoreType.DMA,
        ],
    )(vals, src3, dst3, zeros)


def _dinv_from(dp):
    deg = dp[0, :, 0:1] + dp[1, :, 0:1] + 1.0
    return lax.rsqrt(jnp.maximum(deg, 1.0))


def _tc1_body(x_ref, w_ref, dp_ref, o_ref):
    dinv = _dinv_from(dp_ref[...])
    o_ref[...] = (
        jnp.dot(x_ref[...], w_ref[...], preferred_element_type=jnp.float32) * dinv
    )


def _tc2_body(p_ref, hws_ref, dp_ref, b_ref, w_ref, o_ref):
    dinv = _dinv_from(dp_ref[...])
    p = p_ref[...]
    agg = (p[0] + p[1] + hws_ref[...]) * dinv + b_ref[...]
    h = jnp.maximum(agg, 0.0)
    o_ref[...] = jnp.dot(h, w_ref[...], preferred_element_type=jnp.float32) * dinv


def _tc3_body(p_ref, hws_ref, dp_ref, b_ref, batch_ref, wl_ref, bl_ref, o_ref):
    dinv = _dinv_from(dp_ref[...])
    p = p_ref[...]
    agg = (p[0] + p[1] + hws_ref[...]) * dinv + b_ref[...]
    h = jnp.maximum(agg, 0.0)
    gi = lax.broadcasted_iota(jnp.int32, (_G, 1), 0)
    onehot = (batch_ref[...] == gi).astype(jnp.float32)
    sums = jnp.dot(onehot, h, preferred_element_type=jnp.float32)
    counts = jnp.sum(onehot, axis=1, keepdims=True)
    pooled = sums / jnp.maximum(counts, 1.0)
    o_ref[...] = (
        jnp.dot(pooled, wl_ref[...], preferred_element_type=jnp.float32) + bl_ref[...]
    )


def kernel(x, edge_index, batch, W1, b1, W2, b2, Wl, bl):
    src3 = edge_index[0].reshape(_NC * _NS, _NCH, _KR)
    dst3 = edge_index[1].reshape(_NC * _NS, _NCH, _KR)
    x = jnp.pad(x, ((0, _NP - _N), (0, 0)))
    batch = jnp.pad(batch, (0, _NP - _N), constant_values=_G)
    zeros_rows = jnp.zeros((_NP, _HH), jnp.float32)
    zeros16 = jnp.zeros((_NP, 16), jnp.float32)
    ones16 = jnp.ones((_KR, 16), jnp.float32)

    degp = _deg_call(dst3, ones16, zeros16)

    hws1 = pl.pallas_call(
        _tc1_body, out_shape=jax.ShapeDtypeStruct((_NP, _H), jnp.float32)
    )(x, W1, degp)

    P1 = _row_call(hws1, src3, dst3, zeros_rows)

    hws2 = pl.pallas_call(
        _tc2_body, out_shape=jax.ShapeDtypeStruct((_NP, _H), jnp.float32)
    )(P1, hws1, degp, b1.reshape(1, _H), W2)

    P2 = _row_call(hws2, src3, dst3, zeros_rows)

    out = pl.pallas_call(
        _tc3_body, out_shape=jax.ShapeDtypeStruct((_G, _C), jnp.float32)
    )(P2, hws2, degp, b2.reshape(1, _H), batch.reshape(1, _NP), Wl, bl.reshape(1, _C))
    return out


# speedup vs baseline: 1.7585x; 1.7585x over previous
import functools

import jax
import jax.numpy as jnp
from jax import lax
from jax.experimental import pallas as pl
from jax.experimental.pallas import tpu as pltpu
from jax.experimental.pallas import tpu_sc as plsc

_N = 10000
_E = 320000
_D = 128
_H = 128
_C = 40
_G = 64

_NP = 10240
_NC = 2
_NS = 16
_ROWS_PER_TILE = _NP // _NS
_EDGES_PER_TILE = _E // (_NC * _NS)
_KR = 40
_NCH = _EDGES_PER_TILE // _KR
_NBUF = 5

_MESH = plsc.VectorSubcoreMesh(
    core_axis_name="c", subcore_axis_name="s", num_cores=_NC, num_subcores=_NS
)


def _deg_body(dst_hbm, ones_hbm, zeros_hbm, out_hbm, acc, idx_d, ones_v, sem):
    cid = lax.axis_index("c")
    sid = lax.axis_index("s")
    wid = cid * _NS + sid
    r0 = sid * _ROWS_PER_TILE
    zcp = pltpu.async_copy(
        zeros_hbm.at[pl.ds(r0, _ROWS_PER_TILE)], acc.at[pl.ds(r0, _ROWS_PER_TILE)], sem
    )
    pltpu.sync_copy(dst_hbm.at[wid], idx_d)
    pltpu.sync_copy(ones_hbm, ones_v)
    zcp.wait()
    plsc.subcore_barrier()

    def chunk(j, carry):
        pltpu.sync_copy(ones_v, acc.at[idx_d.at[j]], add=True)
        return carry

    lax.fori_loop(0, _NCH, chunk, 0)
    plsc.subcore_barrier()
    pltpu.sync_copy(
        acc.at[pl.ds(r0, _ROWS_PER_TILE)],
        out_hbm.at[cid, pl.ds(r0, _ROWS_PER_TILE)],
    )


def _row_body(
    vals_hbm, src_hbm, dst_hbm, zeros_hbm, out_hbm,
    acc, idx_s, idx_d, rows0, rows1, rows2, rows3, rows4,
    sem0, sem1, sem2, sem3, sem4,
):
    cid = lax.axis_index("c")
    sid = lax.axis_index("s")
    wid = cid * _NS + sid
    r0 = sid * _ROWS_PER_TILE
    zcp = pltpu.async_copy(
        zeros_hbm.at[pl.ds(r0, _ROWS_PER_TILE)], acc.at[pl.ds(r0, _ROWS_PER_TILE)], sem0
    )
    pltpu.sync_copy(src_hbm.at[wid], idx_s)
    pltpu.sync_copy(dst_hbm.at[wid], idx_d)
    zcp.wait()
    plsc.subcore_barrier()

    bufs = (rows0, rows1, rows2, rows3, rows4)
    sems = (sem0, sem1, sem2, sem3, sem4)
    for b in range(_NBUF - 1):
        pltpu.async_copy(vals_hbm.at[idx_s.at[b]], bufs[b], sems[b])
    ngroups = _NCH // _NBUF

    def group(p, carry):
        j0 = _NBUF * p
        for b in range(_NBUF):
            j = j0 + b
            bn = (b + _NBUF - 1) % _NBUF

            @pl.when(j + _NBUF - 1 < _NCH)
            def _():
                pltpu.async_copy(
                    vals_hbm.at[idx_s.at[j + _NBUF - 1]], bufs[bn], sems[bn]
                )

            pltpu.make_async_copy(vals_hbm.at[idx_s.at[j]], bufs[b], sems[b]).wait()
            pltpu.sync_copy(bufs[b], acc.at[idx_d.at[j]], add=True)
        return carry

    lax.fori_loop(0, ngroups, group, 0)
    plsc.subcore_barrier()
    pltpu.sync_copy(
        acc.at[pl.ds(r0, _ROWS_PER_TILE)],
        out_hbm.at[cid, pl.ds(r0, _ROWS_PER_TILE)],
    )


_SC_PARAMS = pltpu.CompilerParams(use_tc_tiling_on_sc=False)


def _deg_call(dst3, ones16, zeros16):
    return pl.kernel(
        _deg_body,
        out_type=jax.ShapeDtypeStruct((_NC, _NP, 16), jnp.float32),
        mesh=_MESH,
        compiler_params=_SC_PARAMS,
        scratch_types=[
            pltpu.VMEM_SHARED((_NP, 16), jnp.float32),
            pltpu.VMEM((_NCH, _KR), jnp.int32),
            pltpu.VMEM((_KR, 16), jnp.float32),
            pltpu.SemaphoreType.DMA,
        ],
    )(dst3, ones16, zeros16)


def _row_call(vals, src3, dst3, zeros):
    return pl.kernel(
        _row_body,
        out_type=jax.ShapeDtypeStruct((_NC, _NP, _H), jnp.float32),
        mesh=_MESH,
        compiler_params=_SC_PARAMS,
        scratch_types=[
            pltpu.VMEM_SHARED((_NP, _H), jnp.float32),
            pltpu.VMEM((_NCH, _KR), jnp.int32),
            pltpu.VMEM((_NCH, _KR), jnp.int32),
            pltpu.VMEM((_KR, _H), jnp.float32),
            pltpu.VMEM((_KR, _H), jnp.float32),
            pltpu.VMEM((_KR, _H), jnp.float32),
            pltpu.VMEM((_KR, _H), jnp.float32),
            pltpu.VMEM((_KR, _H), jnp.float32),
            pltpu.SemaphoreType.DMA,
            pltpu.SemaphoreType.DMA,
            pltpu.SemaphoreType.DMA,
            pltpu.SemaphoreType.DMA,
            pltpu.SemaphoreType.DMA,
        ],
    )(vals, src3, dst3, zeros)


def _dinv_from(dp):
    deg = dp[0, :, 0:1] + dp[1, :, 0:1] + 1.0
    return lax.rsqrt(jnp.maximum(deg, 1.0))


def _tc1_body(x_ref, w_ref, dp_ref, o_ref):
    dinv = _dinv_from(dp_ref[...])
    o_ref[...] = (
        jnp.dot(x_ref[...], w_ref[...], preferred_element_type=jnp.float32) * dinv
    )


def _tc2_body(p_ref, hws_ref, dp_ref, b_ref, w_ref, o_ref):
    dinv = _dinv_from(dp_ref[...])
    p = p_ref[...]
    agg = (p[0] + p[1] + hws_ref[...]) * dinv + b_ref[...]
    h = jnp.maximum(agg, 0.0)
    o_ref[...] = jnp.dot(h, w_ref[...], preferred_element_type=jnp.float32) * dinv


def _tc3_body(p_ref, hws_ref, dp_ref, b_ref, batch_ref, wl_ref, bl_ref, o_ref):
    dinv = _dinv_from(dp_ref[...])
    p = p_ref[...]
    agg = (p[0] + p[1] + hws_ref[...]) * dinv + b_ref[...]
    h = jnp.maximum(agg, 0.0)
    gi = lax.broadcasted_iota(jnp.int32, (_G, 1), 0)
    onehot = (batch_ref[...] == gi).astype(jnp.float32)
    sums = jnp.dot(onehot, h, preferred_element_type=jnp.float32)
    counts = jnp.sum(onehot, axis=1, keepdims=True)
    pooled = sums / jnp.maximum(counts, 1.0)
    o_ref[...] = (
        jnp.dot(pooled, wl_ref[...], preferred_element_type=jnp.float32) + bl_ref[...]
    )


def kernel(x, edge_index, batch, W1, b1, W2, b2, Wl, bl):
    src3 = edge_index[0].reshape(_NC * _NS, _NCH, _KR)
    dst3 = edge_index[1].reshape(_NC * _NS, _NCH, _KR)
    x = jnp.pad(x, ((0, _NP - _N), (0, 0)))
    batch = jnp.pad(batch, (0, _NP - _N), constant_values=_G)
    zeros_rows = jnp.zeros((_NP, _H), jnp.float32)
    zeros16 = jnp.zeros((_NP, 16), jnp.float32)
    ones16 = jnp.ones((_KR, 16), jnp.float32)

    degp = _deg_call(dst3, ones16, zeros16)

    hws1 = pl.pallas_call(
        _tc1_body, out_shape=jax.ShapeDtypeStruct((_NP, _H), jnp.float32)
    )(x, W1, degp)

    P1 = _row_call(hws1, src3, dst3, zeros_rows)

    hws2 = pl.pallas_call(
        _tc2_body, out_shape=jax.ShapeDtypeStruct((_NP, _H), jnp.float32)
    )(P1, hws1, degp, b1.reshape(1, _H), W2)

    P2 = _row_call(hws2, src3, dst3, zeros_rows)

    out = pl.pallas_call(
        _tc3_body, out_shape=jax.ShapeDtypeStruct((_G, _C), jnp.float32)
    )(P2, hws2, degp, b2.reshape(1, _H), batch.reshape(1, _NP), Wl, bl.reshape(1, _C))
    return out
